# baseline (speedup 1.0000x reference)
import functools

import jax
import jax.numpy as jnp
from jax import lax
from jax.experimental import pallas as pl
from jax.experimental.pallas import tpu as pltpu
from jax.experimental.pallas import tpu_sc as plsc

D_IN_NODE = 64
D_IN_EDGE = 8
D_HID = 300
DP = 384
SL = 128
NS = DP // SL
DEPTH = 3
NB = 16
N = 4096
E = 16384

_mesh = plsc.VectorSubcoreMesh(core_axis_name="c", subcore_axis_name="s")


def _msgpass(hs, es, src2, dst2, zrows):
    CH = 128
    EPT = E // 16
    NCH = EPT // CH
    NPT = N // 16

    @functools.partial(
        pl.kernel,
        out_type=tuple(jax.ShapeDtypeStruct((2 * N, SL), jnp.float32)
                       for _ in range(NS)),
        mesh=_mesh,
        scratch_types=[
            pltpu.VMEM((NCH, CH), jnp.int32),
            pltpu.VMEM((NCH, CH), jnp.int32),
            pltpu.VMEM((CH, SL), jnp.float32),
            pltpu.VMEM((CH, SL), jnp.float32),
            pltpu.VMEM((CH, SL), jnp.float32),
            pltpu.VMEM((CH, SL), jnp.float32),
            pltpu.VMEM((CH, SL), jnp.float32),
            pltpu.VMEM_SHARED((N, SL), jnp.float32),
            pltpu.SemaphoreType.DMA,
            pltpu.SemaphoreType.DMA,
            pltpu.SemaphoreType.DMA,
            pltpu.SemaphoreType.DMA,
            pltpu.SemaphoreType.DMA,
            pltpu.SemaphoreType.DMA,
            pltpu.SemaphoreType.DMA,
        ],
    )
    def k(h0, h1, h2, e0, e1, e2, src_hbm, dst_hbm, zr_hbm, o0, o1, o2,
          src_i, dst_i, rows0, rows1, rows2, ev0, ev1, acc,
          sg0, sg1, sg2, ss0, ss1, ss2, swb):
        cid = lax.axis_index("c")
        sid = lax.axis_index("s")
        wid = cid * 16 + sid
        NR = 3
        rows = (rows0, rows1, rows2)
        ev = (ev0, ev1)
        sg = (sg0, sg1, sg2)
        ss = (ss0, ss1, ss2)

        pltpu.sync_copy(src_hbm.at[pl.ds(wid * NCH, NCH)], src_i)
        pltpu.sync_copy(dst_hbm.at[pl.ds(wid * NCH, NCH)], dst_i)

        ebase = wid * EPT
        stages = ((h0, e0, o0), (h1, e1, o1), (h2, e2, o2))

        def issue(h_hbm, e_hbm, c):
            b = c % NR
            return (pltpu.async_copy(h_hbm.at[src_i.at[c]], rows[b], sg[b]),
                    pltpu.async_copy(e_hbm.at[pl.ds(ebase + c * CH, CH)],
                                     ev[c % 2], sg[b]))

        pltpu.sync_copy(zr_hbm, acc.at[pl.ds(sid * NPT, NPT)])
        pending = issue(h0, e0, 0)
        plsc.subcore_barrier()

        for p, (h_hbm, e_hbm, out_hbm) in enumerate(stages):
            ld = {0: pending}
            sh = {}
            for c in range(NCH):
                b = c % NR
                g, el = ld.pop(c)
                g.wait()
                el.wait()
                if c + 1 < NCH:
                    if c + 1 >= NR:
                        sh[c + 1 - NR].wait()
                    ld[c + 1] = issue(h_hbm, e_hbm, c + 1)

                @plsc.parallel_loop(0, CH, unroll=4)
                def _(r):
                    for j in range(SL // 16):
                        s = pl.ds(j * 16, 16)
                        rows[b][r, s] = jnp.maximum(
                            rows[b][r, s] + ev[c % 2][r, s], 0.0)
                sh[c] = pltpu.async_copy(rows[b], acc.at[dst_i.at[c]], ss[b],
                                         add=True)
            for c in range(NCH - NR, NCH):
                sh[c].wait()
            plsc.subcore_barrier()

            base = sid * NPT
            wb = pltpu.async_copy(acc.at[pl.ds(base, NPT)],
                                  out_hbm.at[pl.ds(cid * N + base, NPT)],
                                  swb)
            wb.wait()
            if p + 1 < NS:
                nh, ne, _ = stages[p + 1]
                pending = issue(nh, ne, 0)
                pltpu.sync_copy(zr_hbm, acc.at[pl.ds(sid * NPT, NPT)])
                plsc.subcore_barrier()

    return k(*hs, *es, src2, dst2, zrows)


def _pool(hs, seg2):
    CH = 128
    NPT = N // 16
    NCH = NPT // CH

    @functools.partial(
        pl.kernel,
        out_type=jax.ShapeDtypeStruct((2, NB, DP), jnp.float32),
        mesh=_mesh,
        scratch_types=[
            pltpu.VMEM((NCH, CH), jnp.int32),
            pltpu.VMEM((CH, SL), jnp.float32),
            pltpu.VMEM_SHARED((NB, SL), jnp.float32),
            pltpu.SemaphoreType.DMA,
        ],
    )
    def k(h0, h1, h2, seg_hbm, out_hbm, seg_i, rows_v, acc, sem):
        cid = lax.axis_index("c")
        sid = lax.axis_index("s")
        wid = cid * 16 + sid

        pltpu.sync_copy(seg_hbm.at[pl.ds(wid * NCH, NCH)], seg_i)

        for p, h_hbm in enumerate((h0, h1, h2)):
            def zrow(r, _):
                for j in range(SL // 16):
                    rows_v[r, pl.ds(j * 16, 16)] = jnp.zeros((16,),
                                                             jnp.float32)
                return 0
            lax.fori_loop(0, NB, zrow, 0)

            @pl.when(sid == 0)
            def _():
                pltpu.sync_copy(rows_v.at[pl.ds(0, NB)], acc)
            plsc.subcore_barrier()

            for t in range(NCH):
                g = cid * N + sid * NPT + t * CH
                pltpu.sync_copy(h_hbm.at[pl.ds(g, CH)], rows_v)
                pltpu.sync_copy(rows_v, acc.at[seg_i.at[t]], add=True)
            plsc.subcore_barrier()

            @pl.when(sid == 0)
            def _():
                pltpu.sync_copy(acc, out_hbm.at[cid, :, pl.ds(p * SL, SL)])
            plsc.subcore_barrier()

    return k(*hs, seg2)


def _init_tc(nf, ef, wn, bn, we, be):
    G = 32
    RN = (2 * N) // G
    RE = (2 * E) // G

    def body(nf_b, wn_b, bn_b, ef_b, we_b, be_b, *outs):
        h = jnp.dot(nf_b[...], wn_b[...], preferred_element_type=jnp.float32)
        h = jnp.maximum(h + bn_b[...], 0.0)
        ee = jnp.dot(ef_b[...], we_b[...], preferred_element_type=jnp.float32)
        ee = ee + be_b[...]
        for p in range(NS):
            outs[p][...] = h[:, p * SL:(p + 1) * SL]
            outs[NS + p][...] = ee[:, p * SL:(p + 1) * SL]

    return pl.pallas_call(
        body,
        grid=(G,),
        in_specs=[
            pl.BlockSpec((RN, D_IN_NODE), lambda i: (i, 0)),
            pl.BlockSpec((D_IN_NODE, DP), lambda i: (0, 0)),
            pl.BlockSpec((1, DP), lambda i: (0, 0)),
            pl.BlockSpec((RE, D_IN_EDGE), lambda i: (i, 0)),
            pl.BlockSpec((D_IN_EDGE, DP), lambda i: (0, 0)),
            pl.BlockSpec((1, DP), lambda i: (0, 0)),
        ],
        out_specs=[pl.BlockSpec((RN, SL), lambda i: (i, 0))] * NS
                  + [pl.BlockSpec((RE, SL), lambda i: (i, 0))] * NS,
        out_shape=[jax.ShapeDtypeStruct((2 * N, SL), jnp.float32)] * NS
                  + [jax.ShapeDtypeStruct((2 * E, SL), jnp.float32)] * NS,
    )(nf, wn, bn, ef, we, be)


def _mlp_tc(hs, aggs, wa, ba_, wb, bb_, relu_out):
    G = 32
    RN = (2 * N) // G

    def body(h0, h1, h2, a0, a1, a2, wa_b, ba_b, wb_b, bb_b, *outs):
        z = jnp.concatenate([h0[...] + a0[...], h1[...] + a1[...],
                             h2[...] + a2[...]], axis=1)
        t = jnp.dot(z, wa_b[...], preferred_element_type=jnp.float32)
        t = jnp.maximum(t + ba_b[...], 0.0)
        o = jnp.dot(t, wb_b[...], preferred_element_type=jnp.float32)
        o = o + bb_b[...]
        if relu_out:
            o = jnp.maximum(o, 0.0)
        for p in range(NS):
            outs[p][...] = o[:, p * SL:(p + 1) * SL]

    return pl.pallas_call(
        body,
        grid=(G,),
        in_specs=[pl.BlockSpec((RN, SL), lambda i: (i, 0))] * (2 * NS) + [
            pl.BlockSpec((DP, DP), lambda i: (0, 0)),
            pl.BlockSpec((1, DP), lambda i: (0, 0)),
            pl.BlockSpec((DP, DP), lambda i: (0, 0)),
            pl.BlockSpec((1, DP), lambda i: (0, 0)),
        ],
        out_specs=[pl.BlockSpec((RN, SL), lambda i: (i, 0))] * NS,
        out_shape=[jax.ShapeDtypeStruct((2 * N, SL), jnp.float32)] * NS,
    )(*hs, *aggs, wa, ba_, wb, bb_)


def kernel(node_feats_r, edge_feats_r, node_feats_p, edge_feats_p, Wn, bn, We,
           be, Wa, ba, Wb, bb, edge_index_r, seg_r, edge_index_p, seg_p):
    PW = DP - D_HID
    nf = jnp.concatenate([node_feats_r, node_feats_p], axis=0)
    ef = jnp.concatenate([edge_feats_r, edge_feats_p], axis=0)
    src = jnp.concatenate([edge_index_r[0], edge_index_p[0] + N])
    dst = jnp.concatenate([edge_index_r[1], edge_index_p[1]])
    src2 = src.astype(jnp.int32).reshape(2 * E // 128, 128)
    dst2 = dst.astype(jnp.int32).reshape(2 * E // 128, 128)
    seg2 = jnp.concatenate([seg_r, seg_p]).astype(jnp.int32).reshape(
        2 * N // 128, 128)

    wn = jnp.pad(Wn, ((0, 0), (0, PW)))
    we = jnp.pad(We, ((0, 0), (0, PW)))
    wa = jnp.pad(Wa, ((0, 0), (0, PW), (0, PW)))
    wb = jnp.pad(Wb, ((0, 0), (0, PW), (0, PW)))
    bn2 = jnp.pad(bn, (0, PW)).reshape(1, DP)
    be2 = jnp.pad(be, (0, PW)).reshape(1, DP)
    ba2 = jnp.pad(ba, ((0, 0), (0, PW)))
    bb2 = jnp.pad(bb, ((0, 0), (0, PW)))

    zrows = jnp.zeros((N // 16, SL), jnp.float32)
    outs = _init_tc(nf, ef, wn, bn2, we, be2)
    hs, es = tuple(outs[:NS]), tuple(outs[NS:])
    for i in range(DEPTH):
        aggs = _msgpass(hs, es, src2, dst2, zrows)
        hs = tuple(_mlp_tc(hs, aggs, wa[i], ba2[i].reshape(1, DP), wb[i],
                           bb2[i].reshape(1, DP), relu_out=(i < DEPTH - 1)))
    pool = _pool(hs, seg2)
    reactants = pool[0, :, :D_HID]
    products = pool[1, :, :D_HID]
    return (reactants - products, reactants, products)


# speedup vs baseline: 1.0422x; 1.0422x over previous
import functools

import jax
import jax.numpy as jnp
from jax import lax
from jax.experimental import pallas as pl
from jax.experimental.pallas import tpu as pltpu
from jax.experimental.pallas import tpu_sc as plsc

D_IN_NODE = 64
D_IN_EDGE = 8
D_HID = 300
DP = 384
SL = 128
NS = DP // SL
DEPTH = 3
NB = 16
N = 4096
E = 16384

_mesh = plsc.VectorSubcoreMesh(core_axis_name="c", subcore_axis_name="s")


def _msgpass(hs, es, src2, dst2):
    CH = 128
    EPT = E // 16
    NCH = EPT // CH
    NPT = N // 16

    @functools.partial(
        pl.kernel,
        out_type=tuple(jax.ShapeDtypeStruct((2 * N, SL), jnp.float32)
                       for _ in range(NS)),
        mesh=_mesh,
        scratch_types=[
            pltpu.VMEM((NCH, CH), jnp.int32),
            pltpu.VMEM((NCH, CH), jnp.int32),
            pltpu.VMEM((CH, SL), jnp.float32),
            pltpu.VMEM((CH, SL), jnp.float32),
            pltpu.VMEM((CH, SL), jnp.float32),
            pltpu.VMEM((CH, SL), jnp.float32),
            pltpu.VMEM_SHARED((N, SL), jnp.float32),
            pltpu.SemaphoreType.DMA,
            pltpu.SemaphoreType.DMA,
            pltpu.SemaphoreType.DMA,
            pltpu.SemaphoreType.DMA,
            pltpu.SemaphoreType.DMA,
            pltpu.SemaphoreType.DMA,
        ],
    )
    def k(h0, h1, h2, e0, e1, e2, src_hbm, dst_hbm, o0, o1, o2,
          src_i, dst_i, rows0, rows1, ev0, ev1, acc,
          sg0, sg1, se0, se1, ss0, ss1):
        cid = lax.axis_index("c")
        sid = lax.axis_index("s")
        wid = cid * 16 + sid
        rows = (rows0, rows1)
        ev = (ev0, ev1)
        sg = (sg0, sg1)
        se = (se0, se1)
        ss = (ss0, ss1)

        pltpu.sync_copy(src_hbm.at[pl.ds(wid * NCH, NCH)], src_i)
        pltpu.sync_copy(dst_hbm.at[pl.ds(wid * NCH, NCH)], dst_i)

        ebase = wid * EPT

        for p, (h_hbm, e_hbm, out_hbm) in enumerate(
                ((h0, e0, o0), (h1, e1, o1), (h2, e2, o2))):
            def zrow(r, _):
                for j in range(SL // 16):
                    ev0[r, pl.ds(j * 16, 16)] = jnp.zeros((16,), jnp.float32)
                return 0
            lax.fori_loop(0, CH, zrow, 0)
            for t in range(NPT // CH):
                pltpu.sync_copy(ev0, acc.at[pl.ds(sid * NPT + t * CH, CH)])
            plsc.subcore_barrier()

            gh = [None] * NCH
            eh = [None] * NCH
            sh = [None] * NCH

            def issue(c):
                b = c & 1
                gh[c] = pltpu.async_copy(h_hbm.at[src_i.at[c]], rows[b],
                                         sg[b])
                eh[c] = pltpu.async_copy(
                    e_hbm.at[pl.ds(ebase + c * CH, CH)], ev[b], se[b])

            issue(0)
            for c in range(NCH):
                b = c & 1
                gh[c].wait()
                eh[c].wait()
                if c + 1 < NCH:
                    if c >= 1:
                        sh[c - 1].wait()
                    issue(c + 1)

                def rrow(r, _):
                    for j in range(SL // 16):
                        s = pl.ds(j * 16, 16)
                        rows[b][r, s] = jnp.maximum(
                            rows[b][r, s] + ev[b][r, s], 0.0)
                    return 0
                lax.fori_loop(0, CH, rrow, 0)
                sh[c] = pltpu.async_copy(rows[b], acc.at[dst_i.at[c]], ss[b],
                                         add=True)
            sh[NCH - 2].wait()
            sh[NCH - 1].wait()
            plsc.subcore_barrier()

            base = sid * NPT
            pltpu.sync_copy(acc.at[pl.ds(base, NPT)],
                            out_hbm.at[pl.ds(cid * N + base, NPT)])
            plsc.subcore_barrier()

    return k(*hs, *es, src2, dst2)


def _pool(hs, seg2):
    CH = 128
    NPT = N // 16
    NCH = NPT // CH

    @functools.partial(
        pl.kernel,
        out_type=jax.ShapeDtypeStruct((2, NB, DP), jnp.float32),
        mesh=_mesh,
        scratch_types=[
            pltpu.VMEM((NCH, CH), jnp.int32),
            pltpu.VMEM((CH, SL), jnp.float32),
            pltpu.VMEM_SHARED((NB, SL), jnp.float32),
            pltpu.SemaphoreType.DMA,
        ],
    )
    def k(h0, h1, h2, seg_hbm, out_hbm, seg_i, rows_v, acc, sem):
        cid = lax.axis_index("c")
        sid = lax.axis_index("s")
        wid = cid * 16 + sid

        pltpu.sync_copy(seg_hbm.at[pl.ds(wid * NCH, NCH)], seg_i)

        for p, h_hbm in enumerate((h0, h1, h2)):
            def zrow(r, _):
                for j in range(SL // 16):
                    rows_v[r, pl.ds(j * 16, 16)] = jnp.zeros((16,),
                                                             jnp.float32)
                return 0
            lax.fori_loop(0, NB, zrow, 0)

            @pl.when(sid == 0)
            def _():
                pltpu.sync_copy(rows_v.at[pl.ds(0, NB)], acc)
            plsc.subcore_barrier()

            for t in range(NCH):
                g = cid * N + sid * NPT + t * CH
                pltpu.sync_copy(h_hbm.at[pl.ds(g, CH)], rows_v)
                pltpu.sync_copy(rows_v, acc.at[seg_i.at[t]], add=True)
            plsc.subcore_barrier()

            @pl.when(sid == 0)
            def _():
                pltpu.sync_copy(acc, out_hbm.at[cid, :, pl.ds(p * SL, SL)])
            plsc.subcore_barrier()

    return k(*hs, seg2)


def _init_tc(nf, ef, wn, bn, we, be):
    G = 32
    RN = (2 * N) // G
    RE = (2 * E) // G

    def body(nf_b, wn_b, bn_b, ef_b, we_b, be_b, *outs):
        h = jnp.dot(nf_b[...], wn_b[...], preferred_element_type=jnp.float32)
        h = jnp.maximum(h + bn_b[...], 0.0)
        ee = jnp.dot(ef_b[...], we_b[...], preferred_element_type=jnp.float32)
        ee = ee + be_b[...]
        for p in range(NS):
            outs[p][...] = h[:, p * SL:(p + 1) * SL]
            outs[NS + p][...] = ee[:, p * SL:(p + 1) * SL]

    return pl.pallas_call(
        body,
        grid=(G,),
        in_specs=[
            pl.BlockSpec((RN, D_IN_NODE), lambda i: (i, 0)),
            pl.BlockSpec((D_IN_NODE, DP), lambda i: (0, 0)),
            pl.BlockSpec((1, DP), lambda i: (0, 0)),
            pl.BlockSpec((RE, D_IN_EDGE), lambda i: (i, 0)),
            pl.BlockSpec((D_IN_EDGE, DP), lambda i: (0, 0)),
            pl.BlockSpec((1, DP), lambda i: (0, 0)),
        ],
        out_specs=[pl.BlockSpec((RN, SL), lambda i: (i, 0))] * NS
                  + [pl.BlockSpec((RE, SL), lambda i: (i, 0))] * NS,
        out_shape=[jax.ShapeDtypeStruct((2 * N, SL), jnp.float32)] * NS
                  + [jax.ShapeDtypeStruct((2 * E, SL), jnp.float32)] * NS,
    )(nf, wn, bn, ef, we, be)


def _mlp_tc(hs, aggs, wa, ba_, wb, bb_, relu_out):
    G = 32
    RN = (2 * N) // G

    def body(h0, h1, h2, a0, a1, a2, wa_b, ba_b, wb_b, bb_b, *outs):
        z = jnp.concatenate([h0[...] + a0[...], h1[...] + a1[...],
                             h2[...] + a2[...]], axis=1)
        t = jnp.dot(z, wa_b[...], preferred_element_type=jnp.float32)
        t = jnp.maximum(t + ba_b[...], 0.0)
        o = jnp.dot(t, wb_b[...], preferred_element_type=jnp.float32)
        o = o + bb_b[...]
        if relu_out:
            o = jnp.maximum(o, 0.0)
        for p in range(NS):
            outs[p][...] = o[:, p * SL:(p + 1) * SL]

    return pl.pallas_call(
        body,
        grid=(G,),
        in_specs=[pl.BlockSpec((RN, SL), lambda i: (i, 0))] * (2 * NS) + [
            pl.BlockSpec((DP, DP), lambda i: (0, 0)),
            pl.BlockSpec((1, DP), lambda i: (0, 0)),
            pl.BlockSpec((DP, DP), lambda i: (0, 0)),
            pl.BlockSpec((1, DP), lambda i: (0, 0)),
        ],
        out_specs=[pl.BlockSpec((RN, SL), lambda i: (i, 0))] * NS,
        out_shape=[jax.ShapeDtypeStruct((2 * N, SL), jnp.float32)] * NS,
    )(*hs, *aggs, wa, ba_, wb, bb_)


def kernel(node_feats_r, edge_feats_r, node_feats_p, edge_feats_p, Wn, bn, We,
           be, Wa, ba, Wb, bb, edge_index_r, seg_r, edge_index_p, seg_p):
    PW = DP - D_HID
    nf = jnp.concatenate([node_feats_r, node_feats_p], axis=0)
    ef = jnp.concatenate([edge_feats_r, edge_feats_p], axis=0)
    src = jnp.concatenate([edge_index_r[0], edge_index_p[0] + N])
    dst = jnp.concatenate([edge_index_r[1], edge_index_p[1]])
    src2 = src.astype(jnp.int32).reshape(2 * E // 128, 128)
    dst2 = dst.astype(jnp.int32).reshape(2 * E // 128, 128)
    seg2 = jnp.concatenate([seg_r, seg_p]).astype(jnp.int32).reshape(
        2 * N // 128, 128)

    wn = jnp.pad(Wn, ((0, 0), (0, PW)))
    we = jnp.pad(We, ((0, 0), (0, PW)))
    wa = jnp.pad(Wa, ((0, 0), (0, PW), (0, PW)))
    wb = jnp.pad(Wb, ((0, 0), (0, PW), (0, PW)))
    bn2 = jnp.pad(bn, (0, PW)).reshape(1, DP)
    be2 = jnp.pad(be, (0, PW)).reshape(1, DP)
    ba2 = jnp.pad(ba, ((0, 0), (0, PW)))
    bb2 = jnp.pad(bb, ((0, 0), (0, PW)))

    outs = _init_tc(nf, ef, wn, bn2, we, be2)
    hs, es = tuple(outs[:NS]), tuple(outs[NS:])
    for i in range(DEPTH):
        aggs = _msgpass(hs, es, src2, dst2)
        hs = tuple(_mlp_tc(hs, aggs, wa[i], ba2[i].reshape(1, DP), wb[i],
                           bb2[i].reshape(1, DP), relu_out=(i < DEPTH - 1)))
    pool = _pool(hs, seg2)
    reactants = pool[0, :, :D_HID]
    products = pool[1, :, :D_HID]
    return (reactants - products, reactants, products)


# speedup vs baseline: 1.0697x; 1.0264x over previous
import functools

import jax
import jax.numpy as jnp
from jax import lax
from jax.experimental import pallas as pl
from jax.experimental.pallas import tpu as pltpu
from jax.experimental.pallas import tpu_sc as plsc

D_IN_NODE = 64
D_IN_EDGE = 8
D_HID = 300
DP = 384
SL = 128
NS = DP // SL
DEPTH = 3
NB = 16
N = 4096
E = 16384

_mesh = plsc.VectorSubcoreMesh(core_axis_name="c", subcore_axis_name="s")


def _msgpass(hs, es, src2, dst2):
    CH = 128
    EPT = E // 32
    NCH = EPT // CH
    NPT = N // 16

    @functools.partial(
        pl.kernel,
        out_type=tuple(jax.ShapeDtypeStruct((2, N, SL), jnp.float32)
                       for _ in range(NS)),
        mesh=_mesh,
        scratch_types=[
            pltpu.VMEM((NCH, CH), jnp.int32),
            pltpu.VMEM((NCH, CH), jnp.int32),
            pltpu.VMEM((CH, SL), jnp.float32),
            pltpu.VMEM((CH, SL), jnp.float32),
            pltpu.VMEM((CH, SL), jnp.float32),
            pltpu.VMEM((CH, SL), jnp.float32),
            pltpu.VMEM_SHARED((N, SL), jnp.float32),
            pltpu.SemaphoreType.DMA,
            pltpu.SemaphoreType.DMA,
            pltpu.SemaphoreType.DMA,
            pltpu.SemaphoreType.DMA,
            pltpu.SemaphoreType.DMA,
            pltpu.SemaphoreType.DMA,
        ],
    )
    def k(h0, h1, h2, e0, e1, e2, src_hbm, dst_hbm, o0, o1, o2,
          src_i, dst_i, rows0, rows1, ev0, ev1, acc,
          sg0, sg1, se0, se1, ss0, ss1):
        cid = lax.axis_index("c")
        sid = lax.axis_index("s")
        wid = cid * 16 + sid
        rows = (rows0, rows1)
        ev = (ev0, ev1)
        sg = (sg0, sg1)
        se = (se0, se1)
        ss = (ss0, ss1)

        pltpu.sync_copy(src_hbm.at[pl.ds(wid * NCH, NCH)], src_i)
        pltpu.sync_copy(dst_hbm.at[pl.ds(wid * NCH, NCH)], dst_i)

        ebase = wid * EPT

        for p, (h_hbm, e_hbm, out_hbm) in enumerate(
                ((h0, e0, o0), (h1, e1, o1), (h2, e2, o2))):
            def zrow(r, _):
                for j in range(SL // 16):
                    ev0[r, pl.ds(j * 16, 16)] = jnp.zeros((16,), jnp.float32)
                return 0
            lax.fori_loop(0, CH, zrow, 0)
            for t in range(NPT // CH):
                pltpu.sync_copy(ev0, acc.at[pl.ds(sid * NPT + t * CH, CH)])
            plsc.subcore_barrier()

            gh = [None] * NCH
            eh = [None] * NCH
            sh = [None] * NCH

            def issue(c):
                b = c & 1
                gh[c] = pltpu.async_copy(h_hbm.at[src_i.at[c]], rows[b],
                                         sg[b])
                eh[c] = pltpu.async_copy(
                    e_hbm.at[pl.ds(ebase + c * CH, CH)], ev[b], se[b])

            issue(0)
            for c in range(NCH):
                b = c & 1
                gh[c].wait()
                eh[c].wait()
                if c + 1 < NCH:
                    if c >= 1:
                        sh[c - 1].wait()
                    issue(c + 1)

                def rrow(r, _):
                    for j in range(SL // 16):
                        s = pl.ds(j * 16, 16)
                        rows[b][r, s] = jnp.maximum(
                            rows[b][r, s] + ev[b][r, s], 0.0)
                    return 0
                lax.fori_loop(0, CH, rrow, 0)
                sh[c] = pltpu.async_copy(rows[b], acc.at[dst_i.at[c]], ss[b],
                                         add=True)
            sh[NCH - 2].wait()
            sh[NCH - 1].wait()
            plsc.subcore_barrier()

            base = sid * NPT
            pltpu.sync_copy(acc.at[pl.ds(base, NPT)],
                            out_hbm.at[cid, pl.ds(base, NPT)])
            plsc.subcore_barrier()

    return k(*hs, *es, src2, dst2)


def _pool(hs, seg2):
    CH = 128

    @functools.partial(
        pl.kernel,
        out_type=jax.ShapeDtypeStruct((2, NB, DP), jnp.float32),
        mesh=_mesh,
        scratch_types=[
            pltpu.VMEM((1, CH), jnp.int32),
            pltpu.VMEM((CH, SL), jnp.float32),
            pltpu.VMEM_SHARED((NB, SL), jnp.float32),
            pltpu.SemaphoreType.DMA,
        ],
    )
    def k(h0, h1, h2, seg_hbm, out_hbm, seg_i, rows_v, acc, sem):
        cid = lax.axis_index("c")
        sid = lax.axis_index("s")
        wid = cid * 16 + sid

        pltpu.sync_copy(seg_hbm.at[pl.ds(wid, 1)], seg_i)

        for p, h_hbm in enumerate((h0, h1, h2)):
            def zrow(r, _):
                for j in range(SL // 16):
                    rows_v[r, pl.ds(j * 16, 16)] = jnp.zeros((16,),
                                                             jnp.float32)
                return 0
            lax.fori_loop(0, NB, zrow, 0)

            @pl.when(sid == 0)
            def _():
                pltpu.sync_copy(rows_v.at[pl.ds(0, NB)], acc)
            plsc.subcore_barrier()

            g = wid * CH
            pltpu.sync_copy(h_hbm.at[pl.ds(g, CH)], rows_v)
            pltpu.sync_copy(rows_v, acc.at[seg_i.at[0]], add=True)
            plsc.subcore_barrier()

            @pl.when(sid == 0)
            def _():
                pltpu.sync_copy(acc, out_hbm.at[cid, :, pl.ds(p * SL, SL)])
            plsc.subcore_barrier()

    return k(*hs, seg2)


def _init_tc(nf, ef, wn, bn, we, be):
    G = 16
    RN = N // G
    RE = E // G

    def body(nf_b, wn_b, bn_b, ef_b, we_b, be_b, *outs):
        h = jnp.dot(nf_b[...], wn_b[...], preferred_element_type=jnp.float32)
        h = jnp.maximum(h + bn_b[...], 0.0)
        ee = jnp.dot(ef_b[...], we_b[...], preferred_element_type=jnp.float32)
        ee = ee + be_b[...]
        for p in range(NS):
            outs[p][...] = h[:, p * SL:(p + 1) * SL]
            outs[NS + p][...] = ee[:, p * SL:(p + 1) * SL]

    return pl.pallas_call(
        body,
        grid=(G,),
        in_specs=[
            pl.BlockSpec((RN, D_IN_NODE), lambda i: (i, 0)),
            pl.BlockSpec((D_IN_NODE, DP), lambda i: (0, 0)),
            pl.BlockSpec((1, DP), lambda i: (0, 0)),
            pl.BlockSpec((RE, D_IN_EDGE), lambda i: (i, 0)),
            pl.BlockSpec((D_IN_EDGE, DP), lambda i: (0, 0)),
            pl.BlockSpec((1, DP), lambda i: (0, 0)),
        ],
        out_specs=[pl.BlockSpec((RN, SL), lambda i: (i, 0))] * NS
                  + [pl.BlockSpec((RE, SL), lambda i: (i, 0))] * NS,
        out_shape=[jax.ShapeDtypeStruct((N, SL), jnp.float32)] * NS
                  + [jax.ShapeDtypeStruct((E, SL), jnp.float32)] * NS,
    )(nf, wn, bn, ef, we, be)


def _mlp_tc(hs, aggs, wa, ba_, wb, bb_, relu_out):
    G = 16
    RN = N // G

    def body(h0, h1, h2, a0, a1, a2, wa_b, ba_b, wb_b, bb_b, *outs):
        z = jnp.concatenate(
            [h0[...] + a0[0] + a0[1], h1[...] + a1[0] + a1[1],
             h2[...] + a2[0] + a2[1]], axis=1)
        t = jnp.dot(z, wa_b[...], preferred_element_type=jnp.float32)
        t = jnp.maximum(t + ba_b[...], 0.0)
        o = jnp.dot(t, wb_b[...], preferred_element_type=jnp.float32)
        o = o + bb_b[...]
        if relu_out:
            o = jnp.maximum(o, 0.0)
        for p in range(NS):
            outs[p][...] = o[:, p * SL:(p + 1) * SL]

    return pl.pallas_call(
        body,
        grid=(G,),
        in_specs=[pl.BlockSpec((RN, SL), lambda i: (i, 0))] * NS
                 + [pl.BlockSpec((2, RN, SL), lambda i: (0, i, 0))] * NS + [
            pl.BlockSpec((DP, DP), lambda i: (0, 0)),
            pl.BlockSpec((1, DP), lambda i: (0, 0)),
            pl.BlockSpec((DP, DP), lambda i: (0, 0)),
            pl.BlockSpec((1, DP), lambda i: (0, 0)),
        ],
        out_specs=[pl.BlockSpec((RN, SL), lambda i: (i, 0))] * NS,
        out_shape=[jax.ShapeDtypeStruct((N, SL), jnp.float32)] * NS,
    )(*hs, *aggs, wa, ba_, wb, bb_)


def kernel(node_feats_r, edge_feats_r, node_feats_p, edge_feats_p, Wn, bn, We,
           be, Wa, ba, Wb, bb, edge_index_r, seg_r, edge_index_p, seg_p):
    PW = DP - D_HID
    i32 = jnp.int32
    srcr = edge_index_r[0].astype(i32).reshape(E // 128, 128)
    dstr = edge_index_r[1].astype(i32).reshape(E // 128, 128)
    srcp = edge_index_p[0].astype(i32).reshape(E // 128, 128)
    dstp = edge_index_p[1].astype(i32).reshape(E // 128, 128)
    segr = seg_r.astype(i32).reshape(N // 128, 128)
    segp = seg_p.astype(i32).reshape(N // 128, 128)

    wn = jnp.pad(Wn, ((0, 0), (0, PW)))
    we = jnp.pad(We, ((0, 0), (0, PW)))
    wa = jnp.pad(Wa, ((0, 0), (0, PW), (0, PW)))
    wb = jnp.pad(Wb, ((0, 0), (0, PW), (0, PW)))
    bn2 = jnp.pad(bn, (0, PW)).reshape(1, DP)
    be2 = jnp.pad(be, (0, PW)).reshape(1, DP)
    ba2 = jnp.pad(ba, ((0, 0), (0, PW)))
    bb2 = jnp.pad(bb, ((0, 0), (0, PW)))

    outs_r = _init_tc(node_feats_r, edge_feats_r, wn, bn2, we, be2)
    outs_p = _init_tc(node_feats_p, edge_feats_p, wn, bn2, we, be2)
    hr, er = tuple(outs_r[:NS]), tuple(outs_r[NS:])
    hp, ep = tuple(outs_p[:NS]), tuple(outs_p[NS:])
    for i in range(DEPTH):
        ar = _msgpass(hr, er, srcr, dstr)
        ap = _msgpass(hp, ep, srcp, dstp)
        relu_out = i < DEPTH - 1
        hr = tuple(_mlp_tc(hr, ar, wa[i], ba2[i].reshape(1, DP), wb[i],
                           bb2[i].reshape(1, DP), relu_out))
        hp = tuple(_mlp_tc(hp, ap, wa[i], ba2[i].reshape(1, DP), wb[i],
                           bb2[i].reshape(1, DP), relu_out))
    pool_r = _pool(hr, segr).sum(axis=0)
    pool_p = _pool(hp, segp).sum(axis=0)
    reactants = pool_r[:, :D_HID]
    products = pool_p[:, :D_HID]
    return (reactants - products, reactants, products)


# speedup vs baseline: 1.0995x; 1.0278x over previous
import functools

import jax
import jax.numpy as jnp
from jax import lax
from jax.experimental import pallas as pl
from jax.experimental.pallas import tpu as pltpu
from jax.experimental.pallas import tpu_sc as plsc

D_IN_NODE = 64
D_IN_EDGE = 8
D_HID = 300
DP = 384
SL = 128
NS = DP // SL
DEPTH = 3
NB = 16
N = 4096
E = 16384

_mesh = plsc.VectorSubcoreMesh(core_axis_name="c", subcore_axis_name="s")


def _msgpass(hs, es, src2, dst2):
    CH = 128
    EPT = E // 32
    NCH = EPT // CH
    NPT = N // 16

    @functools.partial(
        pl.kernel,
        out_type=tuple(jax.ShapeDtypeStruct((2, N, SL), jnp.float32)
                       for _ in range(NS)),
        mesh=_mesh,
        scratch_types=[
            pltpu.VMEM((NCH, CH), jnp.int32),
            pltpu.VMEM((NCH, CH), jnp.int32),
            pltpu.VMEM((CH, SL), jnp.float32),
            pltpu.VMEM((CH, SL), jnp.float32),
            pltpu.VMEM((CH, SL), jnp.float32),
            pltpu.VMEM((CH, SL), jnp.float32),
            pltpu.VMEM((CH, SL), jnp.float32),
            pltpu.VMEM_SHARED((N, SL), jnp.float32),
            pltpu.SemaphoreType.DMA,
            pltpu.SemaphoreType.DMA,
            pltpu.SemaphoreType.DMA,
            pltpu.SemaphoreType.DMA,
            pltpu.SemaphoreType.DMA,
            pltpu.SemaphoreType.DMA,
            pltpu.SemaphoreType.DMA,
            pltpu.SemaphoreType.DMA,
        ],
    )
    def k(h0, h1, h2, e0, e1, e2, src_hbm, dst_hbm, o0, o1, o2,
          src_i, dst_i, rows0, rows1, rows2, ev0, ev1, acc,
          sg0, sg1, sg2, se0, se1, ss0, ss1, ss2):
        cid = lax.axis_index("c")
        sid = lax.axis_index("s")
        wid = cid * 16 + sid
        rows = (rows0, rows1, rows2)
        ev = (ev0, ev1)
        sg = (sg0, sg1, sg2)
        se = (se0, se1)
        ss = (ss0, ss1, ss2)

        pltpu.sync_copy(src_hbm.at[pl.ds(wid * NCH, NCH)], src_i)
        pltpu.sync_copy(dst_hbm.at[pl.ds(wid * NCH, NCH)], dst_i)

        ebase = wid * EPT

        for p, (h_hbm, e_hbm, out_hbm) in enumerate(
                ((h0, e0, o0), (h1, e1, o1), (h2, e2, o2))):
            def zrow(r, _):
                for j in range(SL // 16):
                    ev0[r, pl.ds(j * 16, 16)] = jnp.zeros((16,), jnp.float32)
                return 0
            lax.fori_loop(0, CH, zrow, 0)
            for t in range(NPT // CH):
                pltpu.sync_copy(ev0, acc.at[pl.ds(sid * NPT + t * CH, CH)])
            plsc.subcore_barrier()

            gh = [None] * NCH
            eh = [None] * NCH
            sh = [None] * NCH

            def issue_g(c):
                b = c % 3
                gh[c] = pltpu.async_copy(h_hbm.at[src_i.at[c]], rows[b],
                                         sg[b])

            def issue_e(c):
                eh[c] = pltpu.async_copy(
                    e_hbm.at[pl.ds(ebase + c * CH, CH)], ev[c & 1], se[c & 1])

            issue_g(0)
            issue_g(1)
            issue_e(0)
            issue_e(1)
            for c in range(NCH):
                b = c % 3
                gh[c].wait()
                eh[c].wait()
                if c + 2 < NCH:
                    if c >= 1:
                        sh[c - 1].wait()
                    issue_g(c + 2)

                def rrow(r, _):
                    for j in range(SL // 16):
                        s = pl.ds(j * 16, 16)
                        rows[b][r, s] = jnp.maximum(
                            rows[b][r, s] + ev[c & 1][r, s], 0.0)
                    return 0
                lax.fori_loop(0, CH, rrow, 0)
                if c + 2 < NCH:
                    issue_e(c + 2)
                sh[c] = pltpu.async_copy(rows[b], acc.at[dst_i.at[c]], ss[b],
                                         add=True)
            sh[NCH - 2].wait()
            sh[NCH - 1].wait()
            plsc.subcore_barrier()

            base = sid * NPT
            pltpu.sync_copy(acc.at[pl.ds(base, NPT)],
                            out_hbm.at[cid, pl.ds(base, NPT)])
            plsc.subcore_barrier()

    return k(*hs, *es, src2, dst2)


def _pool(hs, seg2):
    CH = 128

    @functools.partial(
        pl.kernel,
        out_type=jax.ShapeDtypeStruct((2, NB, DP), jnp.float32),
        mesh=_mesh,
        scratch_types=[
            pltpu.VMEM((1, CH), jnp.int32),
            pltpu.VMEM((CH, SL), jnp.float32),
            pltpu.VMEM_SHARED((NB, SL), jnp.float32),
            pltpu.SemaphoreType.DMA,
        ],
    )
    def k(h0, h1, h2, seg_hbm, out_hbm, seg_i, rows_v, acc, sem):
        cid = lax.axis_index("c")
        sid = lax.axis_index("s")
        wid = cid * 16 + sid

        pltpu.sync_copy(seg_hbm.at[pl.ds(wid, 1)], seg_i)

        for p, h_hbm in enumerate((h0, h1, h2)):
            def zrow(r, _):
                for j in range(SL // 16):
                    rows_v[r, pl.ds(j * 16, 16)] = jnp.zeros((16,),
                                                             jnp.float32)
                return 0
            lax.fori_loop(0, NB, zrow, 0)

            @pl.when(sid == 0)
            def _():
                pltpu.sync_copy(rows_v.at[pl.ds(0, NB)], acc)
            plsc.subcore_barrier()

            g = wid * CH
            pltpu.sync_copy(h_hbm.at[pl.ds(g, CH)], rows_v)
            pltpu.sync_copy(rows_v, acc.at[seg_i.at[0]], add=True)
            plsc.subcore_barrier()

            @pl.when(sid == 0)
            def _():
                pltpu.sync_copy(acc, out_hbm.at[cid, :, pl.ds(p * SL, SL)])
            plsc.subcore_barrier()

    return k(*hs, seg2)


def _init_tc(nf, ef, wn, bn, we, be):
    G = 16
    RN = N // G
    RE = E // G

    def body(nf_b, wn_b, bn_b, ef_b, we_b, be_b, *outs):
        h = jnp.dot(nf_b[...], wn_b[...], preferred_element_type=jnp.float32)
        h = jnp.maximum(h + bn_b[...], 0.0)
        ee = jnp.dot(ef_b[...], we_b[...], preferred_element_type=jnp.float32)
        ee = ee + be_b[...]
        for p in range(NS):
            outs[p][...] = h[:, p * SL:(p + 1) * SL]
            outs[NS + p][...] = ee[:, p * SL:(p + 1) * SL]

    return pl.pallas_call(
        body,
        grid=(G,),
        in_specs=[
            pl.BlockSpec((RN, D_IN_NODE), lambda i: (i, 0)),
            pl.BlockSpec((D_IN_NODE, DP), lambda i: (0, 0)),
            pl.BlockSpec((1, DP), lambda i: (0, 0)),
            pl.BlockSpec((RE, D_IN_EDGE), lambda i: (i, 0)),
            pl.BlockSpec((D_IN_EDGE, DP), lambda i: (0, 0)),
            pl.BlockSpec((1, DP), lambda i: (0, 0)),
        ],
        out_specs=[pl.BlockSpec((RN, SL), lambda i: (i, 0))] * NS
                  + [pl.BlockSpec((RE, SL), lambda i: (i, 0))] * NS,
        out_shape=[jax.ShapeDtypeStruct((N, SL), jnp.float32)] * NS
                  + [jax.ShapeDtypeStruct((E, SL), jnp.float32)] * NS,
    )(nf, wn, bn, ef, we, be)


def _mlp_tc(hs, aggs, wa, ba_, wb, bb_, relu_out):
    G = 16
    RN = N // G

    def body(h0, h1, h2, a0, a1, a2, wa_b, ba_b, wb_b, bb_b, *outs):
        z = jnp.concatenate(
            [h0[...] + a0[0] + a0[1], h1[...] + a1[0] + a1[1],
             h2[...] + a2[0] + a2[1]], axis=1)
        t = jnp.dot(z, wa_b[...], preferred_element_type=jnp.float32)
        t = jnp.maximum(t + ba_b[...], 0.0)
        o = jnp.dot(t, wb_b[...], preferred_element_type=jnp.float32)
        o = o + bb_b[...]
        if relu_out:
            o = jnp.maximum(o, 0.0)
        for p in range(NS):
            outs[p][...] = o[:, p * SL:(p + 1) * SL]

    return pl.pallas_call(
        body,
        grid=(G,),
        in_specs=[pl.BlockSpec((RN, SL), lambda i: (i, 0))] * NS
                 + [pl.BlockSpec((2, RN, SL), lambda i: (0, i, 0))] * NS + [
            pl.BlockSpec((DP, DP), lambda i: (0, 0)),
            pl.BlockSpec((1, DP), lambda i: (0, 0)),
            pl.BlockSpec((DP, DP), lambda i: (0, 0)),
            pl.BlockSpec((1, DP), lambda i: (0, 0)),
        ],
        out_specs=[pl.BlockSpec((RN, SL), lambda i: (i, 0))] * NS,
        out_shape=[jax.ShapeDtypeStruct((N, SL), jnp.float32)] * NS,
    )(*hs, *aggs, wa, ba_, wb, bb_)


def kernel(node_feats_r, edge_feats_r, node_feats_p, edge_feats_p, Wn, bn, We,
           be, Wa, ba, Wb, bb, edge_index_r, seg_r, edge_index_p, seg_p):
    PW = DP - D_HID
    i32 = jnp.int32
    srcr = edge_index_r[0].astype(i32).reshape(E // 128, 128)
    dstr = edge_index_r[1].astype(i32).reshape(E // 128, 128)
    srcp = edge_index_p[0].astype(i32).reshape(E // 128, 128)
    dstp = edge_index_p[1].astype(i32).reshape(E // 128, 128)
    segr = seg_r.astype(i32).reshape(N // 128, 128)
    segp = seg_p.astype(i32).reshape(N // 128, 128)

    wn = jnp.pad(Wn, ((0, 0), (0, PW)))
    we = jnp.pad(We, ((0, 0), (0, PW)))
    wa = jnp.pad(Wa, ((0, 0), (0, PW), (0, PW)))
    wb = jnp.pad(Wb, ((0, 0), (0, PW), (0, PW)))
    bn2 = jnp.pad(bn, (0, PW)).reshape(1, DP)
    be2 = jnp.pad(be, (0, PW)).reshape(1, DP)
    ba2 = jnp.pad(ba, ((0, 0), (0, PW)))
    bb2 = jnp.pad(bb, ((0, 0), (0, PW)))

    outs_r = _init_tc(node_feats_r, edge_feats_r, wn, bn2, we, be2)
    outs_p = _init_tc(node_feats_p, edge_feats_p, wn, bn2, we, be2)
    hr, er = tuple(outs_r[:NS]), tuple(outs_r[NS:])
    hp, ep = tuple(outs_p[:NS]), tuple(outs_p[NS:])
    for i in range(DEPTH):
        ar = _msgpass(hr, er, srcr, dstr)
        ap = _msgpass(hp, ep, srcp, dstp)
        relu_out = i < DEPTH - 1
        hr = tuple(_mlp_tc(hr, ar, wa[i], ba2[i].reshape(1, DP), wb[i],
                           bb2[i].reshape(1, DP), relu_out))
        hp = tuple(_mlp_tc(hp, ap, wa[i], ba2[i].reshape(1, DP), wb[i],
                           bb2[i].reshape(1, DP), relu_out))
    pool_r = _pool(hr, segr).sum(axis=0)
    pool_p = _pool(hp, segp).sum(axis=0)
    reactants = pool_r[:, :D_HID]
    products = pool_p[:, :D_HID]
    return (reactants - products, reactants, products)


# speedup vs baseline: 1.1043x; 1.0044x over previous
import functools

import jax
import jax.numpy as jnp
from jax import lax
from jax.experimental import pallas as pl
from jax.experimental.pallas import tpu as pltpu
from jax.experimental.pallas import tpu_sc as plsc

D_IN_NODE = 64
D_IN_EDGE = 8
D_HID = 300
DP = 384
SL = 128
NS = DP // SL
DEPTH = 3
NB = 16
N = 4096
E = 16384

_mesh = plsc.VectorSubcoreMesh(core_axis_name="c", subcore_axis_name="s")


def _msgpass(hs, es, src2, dst2):
    CH = 128
    EPT = E // 32
    NCH = EPT // CH
    NPT = N // 16

    @functools.partial(
        pl.kernel,
        out_type=tuple(jax.ShapeDtypeStruct((2, N, SL), jnp.float32)
                       for _ in range(NS)),
        mesh=_mesh,
        scratch_types=[
            pltpu.VMEM((NCH, CH), jnp.int32),
            pltpu.VMEM((NCH, CH), jnp.int32),
            pltpu.VMEM((CH, SL), jnp.float32),
            pltpu.VMEM((CH, SL), jnp.float32),
            pltpu.VMEM((CH, SL), jnp.float32),
            pltpu.VMEM((CH, SL), jnp.float32),
            pltpu.VMEM((CH, SL), jnp.float32),
            pltpu.VMEM_SHARED((N, SL), jnp.float32),
            pltpu.SemaphoreType.DMA,
            pltpu.SemaphoreType.DMA,
            pltpu.SemaphoreType.DMA,
            pltpu.SemaphoreType.DMA,
            pltpu.SemaphoreType.DMA,
            pltpu.SemaphoreType.DMA,
            pltpu.SemaphoreType.DMA,
            pltpu.SemaphoreType.DMA,
        ],
    )
    def k(h0, h1, h2, e0, e1, e2, src_hbm, dst_hbm, o0, o1, o2,
          src_i, dst_i, rows0, rows1, rows2, ev0, ev1, acc,
          sg0, sg1, sg2, se0, se1, ss0, ss1, ss2):
        cid = lax.axis_index("c")
        sid = lax.axis_index("s")
        wid = cid * 16 + sid
        rows = (rows0, rows1, rows2)
        ev = (ev0, ev1)
        sg = (sg0, sg1, sg2)
        se = (se0, se1)
        ss = (ss0, ss1, ss2)

        pltpu.sync_copy(src_hbm.at[pl.ds(wid * NCH, NCH)], src_i)
        pltpu.sync_copy(dst_hbm.at[pl.ds(wid * NCH, NCH)], dst_i)

        ebase = wid * EPT

        for p, (h_hbm, e_hbm, out_hbm) in enumerate(
                ((h0, e0, o0), (h1, e1, o1), (h2, e2, o2))):
            def zrow(r, _):
                for j in range(SL // 16):
                    ev0[r, pl.ds(j * 16, 16)] = jnp.zeros((16,), jnp.float32)
                return 0
            lax.fori_loop(0, CH, zrow, 0)
            for t in range(NPT // CH):
                pltpu.sync_copy(ev0, acc.at[pl.ds(sid * NPT + t * CH, CH)])
            plsc.subcore_barrier()

            gh = [None] * NCH
            eh = [None] * NCH
            sh = [None] * NCH

            def issue_g(c):
                b = c % 3
                gh[c] = pltpu.async_copy(h_hbm.at[src_i.at[c]], rows[b],
                                         sg[b])

            def issue_e(c):
                eh[c] = pltpu.async_copy(
                    e_hbm.at[pl.ds(ebase + c * CH, CH)], ev[c & 1], se[c & 1])

            issue_g(0)
            issue_g(1)
            issue_e(0)
            issue_e(1)
            for c in range(NCH):
                b = c % 3
                gh[c].wait()
                eh[c].wait()
                if c + 2 < NCH:
                    if c >= 1:
                        sh[c - 1].wait()
                    issue_g(c + 2)

                def rrow(r, _):
                    for j in range(SL // 16):
                        s = pl.ds(j * 16, 16)
                        rows[b][r, s] = jnp.maximum(
                            rows[b][r, s] + ev[c & 1][r, s], 0.0)
                    return 0
                lax.fori_loop(0, CH, rrow, 0)
                if c + 2 < NCH:
                    issue_e(c + 2)
                sh[c] = pltpu.async_copy(rows[b], acc.at[dst_i.at[c]], ss[b],
                                         add=True)
            sh[NCH - 2].wait()
            sh[NCH - 1].wait()
            plsc.subcore_barrier()

            base = sid * NPT
            pltpu.sync_copy(acc.at[pl.ds(base, NPT)],
                            out_hbm.at[cid, pl.ds(base, NPT)])

    return k(*hs, *es, src2, dst2)


def _pool(hs, seg2):
    CH = 128

    @functools.partial(
        pl.kernel,
        out_type=jax.ShapeDtypeStruct((2, NB, DP), jnp.float32),
        mesh=_mesh,
        scratch_types=[
            pltpu.VMEM((1, CH), jnp.int32),
            pltpu.VMEM((CH, SL), jnp.float32),
            pltpu.VMEM((CH, SL), jnp.float32),
            pltpu.VMEM((CH, SL), jnp.float32),
            pltpu.VMEM_SHARED((NB, SL), jnp.float32),
            pltpu.VMEM_SHARED((NB, SL), jnp.float32),
            pltpu.VMEM_SHARED((NB, SL), jnp.float32),
            pltpu.SemaphoreType.DMA,
            pltpu.SemaphoreType.DMA,
            pltpu.SemaphoreType.DMA,
        ],
    )
    def k(h0, h1, h2, seg_hbm, out_hbm, seg_i, r0, r1, r2,
          acc0, acc1, acc2, s0, s1, s2):
        cid = lax.axis_index("c")
        sid = lax.axis_index("s")
        wid = cid * 16 + sid
        rv = (r0, r1, r2)
        accs = (acc0, acc1, acc2)
        sems = (s0, s1, s2)

        pltpu.sync_copy(seg_hbm.at[pl.ds(wid, 1)], seg_i)

        def zrow(r, _):
            for j in range(SL // 16):
                r0[r, pl.ds(j * 16, 16)] = jnp.zeros((16,), jnp.float32)
            return 0
        lax.fori_loop(0, NB, zrow, 0)

        @pl.when(sid == 0)
        def _():
            for p in range(NS):
                pltpu.sync_copy(r0.at[pl.ds(0, NB)], accs[p])
        plsc.subcore_barrier()

        g = wid * CH
        hl = [pltpu.async_copy(h_hbm.at[pl.ds(g, CH)], rv[p], sems[p])
              for p, h_hbm in enumerate((h0, h1, h2))]
        for p in range(NS):
            hl[p].wait()
            pltpu.sync_copy(rv[p], accs[p].at[seg_i.at[0]], add=True)
        plsc.subcore_barrier()

        @pl.when(sid == 0)
        def _():
            for p in range(NS):
                pltpu.sync_copy(accs[p],
                                out_hbm.at[cid, :, pl.ds(p * SL, SL)])

    return k(*hs, seg2)


def _init_tc(nf, ef, wn, bn, we, be):
    G = 16
    RN = N // G
    RE = E // G

    def body(nf_b, wn_b, bn_b, ef_b, we_b, be_b, *outs):
        h = jnp.dot(nf_b[...], wn_b[...], preferred_element_type=jnp.float32)
        h = jnp.maximum(h + bn_b[...], 0.0)
        ee = jnp.dot(ef_b[...], we_b[...], preferred_element_type=jnp.float32)
        ee = ee + be_b[...]
        for p in range(NS):
            outs[p][...] = h[:, p * SL:(p + 1) * SL]
            outs[NS + p][...] = ee[:, p * SL:(p + 1) * SL]

    return pl.pallas_call(
        body,
        grid=(G,),
        in_specs=[
            pl.BlockSpec((RN, D_IN_NODE), lambda i: (i, 0)),
            pl.BlockSpec((D_IN_NODE, DP), lambda i: (0, 0)),
            pl.BlockSpec((1, DP), lambda i: (0, 0)),
            pl.BlockSpec((RE, D_IN_EDGE), lambda i: (i, 0)),
            pl.BlockSpec((D_IN_EDGE, DP), lambda i: (0, 0)),
            pl.BlockSpec((1, DP), lambda i: (0, 0)),
        ],
        out_specs=[pl.BlockSpec((RN, SL), lambda i: (i, 0))] * NS
                  + [pl.BlockSpec((RE, SL), lambda i: (i, 0))] * NS,
        out_shape=[jax.ShapeDtypeStruct((N, SL), jnp.float32)] * NS
                  + [jax.ShapeDtypeStruct((E, SL), jnp.float32)] * NS,
    )(nf, wn, bn, ef, we, be)


def _mlp_tc(hs, aggs, wa, ba_, wb, bb_, relu_out):
    G = 16
    RN = N // G

    def body(h0, h1, h2, a0, a1, a2, wa_b, ba_b, wb_b, bb_b, *outs):
        z = jnp.concatenate(
            [h0[...] + a0[0] + a0[1], h1[...] + a1[0] + a1[1],
             h2[...] + a2[0] + a2[1]], axis=1)
        t = jnp.dot(z, wa_b[...], preferred_element_type=jnp.float32)
        t = jnp.maximum(t + ba_b[...], 0.0)
        o = jnp.dot(t, wb_b[...], preferred_element_type=jnp.float32)
        o = o + bb_b[...]
        if relu_out:
            o = jnp.maximum(o, 0.0)
        for p in range(NS):
            outs[p][...] = o[:, p * SL:(p + 1) * SL]

    return pl.pallas_call(
        body,
        grid=(G,),
        in_specs=[pl.BlockSpec((RN, SL), lambda i: (i, 0))] * NS
                 + [pl.BlockSpec((2, RN, SL), lambda i: (0, i, 0))] * NS + [
            pl.BlockSpec((DP, DP), lambda i: (0, 0)),
            pl.BlockSpec((1, DP), lambda i: (0, 0)),
            pl.BlockSpec((DP, DP), lambda i: (0, 0)),
            pl.BlockSpec((1, DP), lambda i: (0, 0)),
        ],
        out_specs=[pl.BlockSpec((RN, SL), lambda i: (i, 0))] * NS,
        out_shape=[jax.ShapeDtypeStruct((N, SL), jnp.float32)] * NS,
    )(*hs, *aggs, wa, ba_, wb, bb_)


def kernel(node_feats_r, edge_feats_r, node_feats_p, edge_feats_p, Wn, bn, We,
           be, Wa, ba, Wb, bb, edge_index_r, seg_r, edge_index_p, seg_p):
    PW = DP - D_HID
    i32 = jnp.int32
    srcr = edge_index_r[0].astype(i32).reshape(E // 128, 128)
    dstr = edge_index_r[1].astype(i32).reshape(E // 128, 128)
    srcp = edge_index_p[0].astype(i32).reshape(E // 128, 128)
    dstp = edge_index_p[1].astype(i32).reshape(E // 128, 128)
    segr = seg_r.astype(i32).reshape(N // 128, 128)
    segp = seg_p.astype(i32).reshape(N // 128, 128)

    wn = jnp.pad(Wn, ((0, 0), (0, PW)))
    we = jnp.pad(We, ((0, 0), (0, PW)))
    wa = jnp.pad(Wa, ((0, 0), (0, PW), (0, PW)))
    wb = jnp.pad(Wb, ((0, 0), (0, PW), (0, PW)))
    bn2 = jnp.pad(bn, (0, PW)).reshape(1, DP)
    be2 = jnp.pad(be, (0, PW)).reshape(1, DP)
    ba2 = jnp.pad(ba, ((0, 0), (0, PW)))
    bb2 = jnp.pad(bb, ((0, 0), (0, PW)))

    outs_r = _init_tc(node_feats_r, edge_feats_r, wn, bn2, we, be2)
    outs_p = _init_tc(node_feats_p, edge_feats_p, wn, bn2, we, be2)
    hr, er = tuple(outs_r[:NS]), tuple(outs_r[NS:])
    hp, ep = tuple(outs_p[:NS]), tuple(outs_p[NS:])
    for i in range(DEPTH):
        ar = _msgpass(hr, er, srcr, dstr)
        ap = _msgpass(hp, ep, srcp, dstp)
        relu_out = i < DEPTH - 1
        hr = tuple(_mlp_tc(hr, ar, wa[i], ba2[i].reshape(1, DP), wb[i],
                           bb2[i].reshape(1, DP), relu_out))
        hp = tuple(_mlp_tc(hp, ap, wa[i], ba2[i].reshape(1, DP), wb[i],
                           bb2[i].reshape(1, DP), relu_out))
    pool_r = _pool(hr, segr).sum(axis=0)
    pool_p = _pool(hp, segp).sum(axis=0)
    reactants = pool_r[:, :D_HID]
    products = pool_p[:, :D_HID]
    return (reactants - products, reactants, products)


# speedup vs baseline: 1.1056x; 1.0011x over previous
import functools

import jax
import jax.numpy as jnp
from jax import lax
from jax.experimental import pallas as pl
from jax.experimental.pallas import tpu as pltpu
from jax.experimental.pallas import tpu_sc as plsc

D_IN_NODE = 64
D_IN_EDGE = 8
D_HID = 300
DP = 384
SL = 128
NS = DP // SL
DEPTH = 3
NB = 16
N = 4096
E = 16384

_mesh = plsc.VectorSubcoreMesh(core_axis_name="c", subcore_axis_name="s")


def _msgpass(hs, es, src2, dst2):
    CH = 128
    EPT = E // 32
    NCH = EPT // CH
    NPT = N // 16

    @functools.partial(
        pl.kernel,
        out_type=tuple(jax.ShapeDtypeStruct((2, N, SL), jnp.float32)
                       for _ in range(NS)),
        mesh=_mesh,
        scratch_types=[
            pltpu.VMEM((NCH, CH), jnp.int32),
            pltpu.VMEM((NCH, CH), jnp.int32),
            pltpu.VMEM((CH, SL), jnp.float32),
            pltpu.VMEM((CH, SL), jnp.float32),
            pltpu.VMEM((CH, SL), jnp.float32),
            pltpu.VMEM((CH, SL), jnp.float32),
            pltpu.VMEM((CH, SL), jnp.float32),
            pltpu.VMEM_SHARED((N, SL), jnp.float32),
            pltpu.SemaphoreType.DMA,
            pltpu.SemaphoreType.DMA,
            pltpu.SemaphoreType.DMA,
            pltpu.SemaphoreType.DMA,
            pltpu.SemaphoreType.DMA,
            pltpu.SemaphoreType.DMA,
            pltpu.SemaphoreType.DMA,
            pltpu.SemaphoreType.DMA,
        ],
    )
    def k(h0, h1, h2, e0, e1, e2, src_hbm, dst_hbm, o0, o1, o2,
          src_i, dst_i, rows0, rows1, rows2, ev0, ev1, acc,
          sg0, sg1, sg2, se0, se1, ss0, ss1, ss2):
        cid = lax.axis_index("c")
        sid = lax.axis_index("s")
        wid = cid * 16 + sid
        rows = (rows0, rows1, rows2)
        ev = (ev0, ev1)
        sg = (sg0, sg1, sg2)
        se = (se0, se1)
        ss = (ss0, ss1, ss2)

        pltpu.sync_copy(src_hbm.at[pl.ds(wid * NCH, NCH)], src_i)
        pltpu.sync_copy(dst_hbm.at[pl.ds(wid * NCH, NCH)], dst_i)

        ebase = wid * EPT

        for p, (h_hbm, e_hbm, out_hbm) in enumerate(
                ((h0, e0, o0), (h1, e1, o1), (h2, e2, o2))):
            def zrow(r, _):
                for j in range(SL // 16):
                    ev0[r, pl.ds(j * 16, 16)] = jnp.zeros((16,), jnp.float32)
                return 0
            lax.fori_loop(0, CH, zrow, 0)
            for t in range(NPT // CH):
                pltpu.sync_copy(ev0, acc.at[pl.ds(sid * NPT + t * CH, CH)])
            plsc.subcore_barrier()

            gh = [None] * NCH
            eh = [None] * NCH
            sh = [None] * NCH

            def issue_g(c):
                b = c % 3
                gh[c] = pltpu.async_copy(h_hbm.at[src_i.at[c]], rows[b],
                                         sg[b])

            def issue_e(c):
                eh[c] = pltpu.async_copy(
                    e_hbm.at[pl.ds(ebase + c * CH, CH)], ev[c & 1], se[c & 1])

            issue_g(0)
            issue_g(1)
            issue_e(0)
            issue_e(1)
            for c in range(NCH):
                b = c % 3
                gh[c].wait()
                eh[c].wait()
                if c + 2 < NCH:
                    if c >= 1:
                        sh[c - 1].wait()
                    issue_g(c + 2)

                def rrow(r, _):
                    for j in range(SL // 16):
                        s = pl.ds(j * 16, 16)
                        rows[b][r, s] = jnp.maximum(
                            rows[b][r, s] + ev[c & 1][r, s], 0.0)
                    return 0
                lax.fori_loop(0, CH, rrow, 0)
                if c + 2 < NCH:
                    issue_e(c + 2)
                sh[c] = pltpu.async_copy(rows[b], acc.at[dst_i.at[c]], ss[b],
                                         add=True)
            waited = {c - 1 for c in range(NCH) if c + 2 < NCH and c >= 1}
            for c in range(NCH):
                if c not in waited:
                    sh[c].wait()
            plsc.subcore_barrier()

            base = sid * NPT
            pltpu.sync_copy(acc.at[pl.ds(base, NPT)],
                            out_hbm.at[cid, pl.ds(base, NPT)])

    return k(*hs, *es, src2, dst2)


def _pool(hs, seg2):
    CH = 128

    @functools.partial(
        pl.kernel,
        out_type=jax.ShapeDtypeStruct((2, NB, DP), jnp.float32),
        mesh=_mesh,
        scratch_types=[
            pltpu.VMEM((1, CH), jnp.int32),
            pltpu.VMEM((CH, SL), jnp.float32),
            pltpu.VMEM((CH, SL), jnp.float32),
            pltpu.VMEM((CH, SL), jnp.float32),
            pltpu.VMEM_SHARED((NB, SL), jnp.float32),
            pltpu.VMEM_SHARED((NB, SL), jnp.float32),
            pltpu.VMEM_SHARED((NB, SL), jnp.float32),
            pltpu.SemaphoreType.DMA,
            pltpu.SemaphoreType.DMA,
            pltpu.SemaphoreType.DMA,
        ],
    )
    def k(h0, h1, h2, seg_hbm, out_hbm, seg_i, r0, r1, r2,
          acc0, acc1, acc2, s0, s1, s2):
        cid = lax.axis_index("c")
        sid = lax.axis_index("s")
        wid = cid * 16 + sid
        rv = (r0, r1, r2)
        accs = (acc0, acc1, acc2)
        sems = (s0, s1, s2)

        pltpu.sync_copy(seg_hbm.at[pl.ds(wid, 1)], seg_i)

        def zrow(r, _):
            for j in range(SL // 16):
                r0[r, pl.ds(j * 16, 16)] = jnp.zeros((16,), jnp.float32)
            return 0
        lax.fori_loop(0, NB, zrow, 0)

        @pl.when(sid == 0)
        def _():
            for p in range(NS):
                pltpu.sync_copy(r0.at[pl.ds(0, NB)], accs[p])
        plsc.subcore_barrier()

        g = wid * CH
        hl = [pltpu.async_copy(h_hbm.at[pl.ds(g, CH)], rv[p], sems[p])
              for p, h_hbm in enumerate((h0, h1, h2))]
        for p in range(NS):
            hl[p].wait()
            pltpu.sync_copy(rv[p], accs[p].at[seg_i.at[0]], add=True)
        plsc.subcore_barrier()

        @pl.when(sid == 0)
        def _():
            for p in range(NS):
                pltpu.sync_copy(accs[p],
                                out_hbm.at[cid, :, pl.ds(p * SL, SL)])

    return k(*hs, seg2)


def _init_tc(nf, ef, wn, bn, we, be):
    G = 16
    RN = N // G
    RE = E // G

    def body(nf_b, wn_b, bn_b, ef_b, we_b, be_b, *outs):
        h = jnp.dot(nf_b[...], wn_b[...], preferred_element_type=jnp.float32)
        h = jnp.maximum(h + bn_b[...], 0.0)
        ee = jnp.dot(ef_b[...], we_b[...], preferred_element_type=jnp.float32)
        ee = ee + be_b[...]
        for p in range(NS):
            outs[p][...] = h[:, p * SL:(p + 1) * SL]
            outs[NS + p][...] = ee[:, p * SL:(p + 1) * SL]

    return pl.pallas_call(
        body,
        grid=(G,),
        in_specs=[
            pl.BlockSpec((RN, D_IN_NODE), lambda i: (i, 0)),
            pl.BlockSpec((D_IN_NODE, DP), lambda i: (0, 0)),
            pl.BlockSpec((1, DP), lambda i: (0, 0)),
            pl.BlockSpec((RE, D_IN_EDGE), lambda i: (i, 0)),
            pl.BlockSpec((D_IN_EDGE, DP), lambda i: (0, 0)),
            pl.BlockSpec((1, DP), lambda i: (0, 0)),
        ],
        out_specs=[pl.BlockSpec((RN, SL), lambda i: (i, 0))] * NS
                  + [pl.BlockSpec((RE, SL), lambda i: (i, 0))] * NS,
        out_shape=[jax.ShapeDtypeStruct((N, SL), jnp.float32)] * NS
                  + [jax.ShapeDtypeStruct((E, SL), jnp.float32)] * NS,
    )(nf, wn, bn, ef, we, be)


def _mlp_tc(hs, aggs, wa, ba_, wb, bb_, relu_out):
    G = 16
    RN = N // G

    def body(h0, h1, h2, a0, a1, a2, wa_b, ba_b, wb_b, bb_b, *outs):
        z = jnp.concatenate(
            [h0[...] + a0[0] + a0[1], h1[...] + a1[0] + a1[1],
             h2[...] + a2[0] + a2[1]], axis=1)
        t = jnp.dot(z, wa_b[...], preferred_element_type=jnp.float32)
        t = jnp.maximum(t + ba_b[...], 0.0)
        o = jnp.dot(t, wb_b[...], preferred_element_type=jnp.float32)
        o = o + bb_b[...]
        if relu_out:
            o = jnp.maximum(o, 0.0)
        for p in range(NS):
            outs[p][...] = o[:, p * SL:(p + 1) * SL]

    return pl.pallas_call(
        body,
        grid=(G,),
        in_specs=[pl.BlockSpec((RN, SL), lambda i: (i, 0))] * NS
                 + [pl.BlockSpec((2, RN, SL), lambda i: (0, i, 0))] * NS + [
            pl.BlockSpec((DP, DP), lambda i: (0, 0)),
            pl.BlockSpec((1, DP), lambda i: (0, 0)),
            pl.BlockSpec((DP, DP), lambda i: (0, 0)),
            pl.BlockSpec((1, DP), lambda i: (0, 0)),
        ],
        out_specs=[pl.BlockSpec((RN, SL), lambda i: (i, 0))] * NS,
        out_shape=[jax.ShapeDtypeStruct((N, SL), jnp.float32)] * NS,
    )(*hs, *aggs, wa, ba_, wb, bb_)


def kernel(node_feats_r, edge_feats_r, node_feats_p, edge_feats_p, Wn, bn, We,
           be, Wa, ba, Wb, bb, edge_index_r, seg_r, edge_index_p, seg_p):
    PW = DP - D_HID
    i32 = jnp.int32
    srcr = edge_index_r[0].astype(i32).reshape(E // 128, 128)
    dstr = edge_index_r[1].astype(i32).reshape(E // 128, 128)
    srcp = edge_index_p[0].astype(i32).reshape(E // 128, 128)
    dstp = edge_index_p[1].astype(i32).reshape(E // 128, 128)
    segr = seg_r.astype(i32).reshape(N // 128, 128)
    segp = seg_p.astype(i32).reshape(N // 128, 128)

    wn = jnp.pad(Wn, ((0, 0), (0, PW)))
    we = jnp.pad(We, ((0, 0), (0, PW)))
    wa = jnp.pad(Wa, ((0, 0), (0, PW), (0, PW)))
    wb = jnp.pad(Wb, ((0, 0), (0, PW), (0, PW)))
    bn2 = jnp.pad(bn, (0, PW)).reshape(1, DP)
    be2 = jnp.pad(be, (0, PW)).reshape(1, DP)
    ba2 = jnp.pad(ba, ((0, 0), (0, PW)))
    bb2 = jnp.pad(bb, ((0, 0), (0, PW)))

    outs_r = _init_tc(node_feats_r, edge_feats_r, wn, bn2, we, be2)
    outs_p = _init_tc(node_feats_p, edge_feats_p, wn, bn2, we, be2)
    hr, er = tuple(outs_r[:NS]), tuple(outs_r[NS:])
    hp, ep = tuple(outs_p[:NS]), tuple(outs_p[NS:])
    for i in range(DEPTH):
        ar = _msgpass(hr, er, srcr, dstr)
        ap = _msgpass(hp, ep, srcp, dstp)
        relu_out = i < DEPTH - 1
        hr = tuple(_mlp_tc(hr, ar, wa[i], ba2[i].reshape(1, DP), wb[i],
                           bb2[i].reshape(1, DP), relu_out))
        hp = tuple(_mlp_tc(hp, ap, wa[i], ba2[i].reshape(1, DP), wb[i],
                           bb2[i].reshape(1, DP), relu_out))
    pool_r = _pool(hr, segr).sum(axis=0)
    pool_p = _pool(hp, segp).sum(axis=0)
    reactants = pool_r[:, :D_HID]
    products = pool_p[:, :D_HID]
    return (reactants - products, reactants, products)
